# baseline (speedup 1.0000x reference)
import functools

import jax
import jax.numpy as jnp
from jax import lax
from jax.experimental import pallas as pl
from jax.experimental.pallas import tpu as pltpu
from jax.experimental.pallas import tpu_sc as plsc

_V = 10000
_D = 256
_B = 16384
_S = 200

_NC, _NS, _L = 2, 16, 16
_NW = _NC * _NS
_RPW = _B // _NW
_G = 16
_NG = _RPW // _G
_CH = 4
_SC = _S // _CH
_DV = _D // _L
_PK = _D // (2 * _L)
_TCH = 5


def _rsqrt_nr(x):
    xi = plsc.bitcast(x, jnp.int32)
    yi = jnp.int32(0x5F3759DF) - (xi >> 1)
    y = plsc.bitcast(yi, jnp.float32)
    for _ in range(3):
        y = y * (1.5 - 0.5 * x * y * y)
    return y


def _sc_body(pred_hbm, gt_hbm, table_hbm, out_hbm,
             idx_p, idx_g, bufs, tshared, sums_p, sums_g, rewards,
             sem0, sem1, sem2, sem3):
    sid = lax.axis_index("s")
    wid = sid * _NC + lax.axis_index("c")
    base = wid * _RPW
    zero = jnp.zeros((_L,), jnp.float32)
    rows16 = lax.iota(jnp.int32, _L) * _D
    sems = ((sem0, sem1), (sem2, sem3))

    @pl.when(sid == 0)
    def _():
        pltpu.sync_copy(table_hbm, tshared)
    plsc.subcore_barrier()

    def chunk_copies(i, c, par):
        return [pltpu.make_async_copy(
                    tbl.at[idx.at[i, c]], bufs.at[par, side],
                    sems[par][side])
                for side, idx, tbl in ((0, idx_p, tshared),
                                       (1, idx_g, table_hbm))]

    def issue_chunk(i, c, par):
        for cp in chunk_copies(i, c, par):
            cp.start()

    def wait_chunk(i, c, par):
        for cp in chunk_copies(i, c, par):
            cp.wait()

    def reduce_chunk(par, accs):
        def run(jj, accs_):
            f = list(accs_)
            j0 = jj * _TCH
            for side in range(2):
                for k in range(_PK):
                    b = bufs[par, side, j0, pl.ds(k * 2 * _L, 2 * _L)]
                    for t in range(1, _TCH):
                        b = b + bufs[par, side, j0 + t, pl.ds(k * 2 * _L, 2 * _L)]
                    lo, hi = plsc.unpack(b, format=plsc.PackFormat.INTERLEAVED)
                    f[side * _DV + 2 * k] += lo
                    f[side * _DV + 2 * k + 1] += hi
            return tuple(f)

        return lax.fori_loop(0, _SC // _TCH, run, accs)

    def group_body(g, carry):
        rbase = base + g * _G
        pltpu.sync_copy(pred_hbm.at[pl.ds(rbase, _G)], idx_p)
        pltpu.sync_copy(gt_hbm.at[pl.ds(rbase, _G)], idx_g)
        issue_chunk(0, 0, 0)

        def row_body(i, c2):
            accs = (zero,) * (2 * _DV)
            for c in range(_CH):
                par = c % 2
                if c + 1 < _CH:
                    issue_chunk(i, c + 1, 1 - par)
                else:
                    @pl.when(i + 1 < _G)
                    def _():
                        issue_chunk(i + 1, 0, 1 - par)
                wait_chunk(i, c, par)
                accs = reduce_chunk(par, accs)
            for k in range(_DV):
                sums_p[pl.ds(i * _D + k * _L, _L)] = accs[k]
                sums_g[pl.ds(i * _D + k * _L, _L)] = accs[_DV + k]
            return c2

        lax.fori_loop(0, _G, row_body, 0)

        def fin(d, carry3):
            dot, np_, ng_ = carry3
            idxv = rows16 + d
            p = plsc.load_gather(sums_p, [idxv])
            q = plsc.load_gather(sums_g, [idxv])
            return dot + p * q, np_ + p * p, ng_ + q * q

        dot, np_, ng_ = lax.fori_loop(0, _D, fin, (zero, zero, zero))
        inv2 = jnp.float32(1.0 / (_S * _S))
        np_m = jnp.maximum(np_ * inv2, 1e-16)
        ng_m = jnp.maximum(ng_ * inv2, 1e-16)
        sim = dot * inv2 * _rsqrt_nr(np_m * ng_m)
        rewards[pl.ds(g * _G, _G)] = 0.7 * jnp.maximum(sim, 0.0)
        return carry

    lax.fori_loop(0, _NG, group_body, 0)
    pltpu.sync_copy(rewards, out_hbm.at[pl.ds(base, _RPW)])


def _make_sc_kernel(interpret=False):
    mesh = plsc.VectorSubcoreMesh(core_axis_name="c", subcore_axis_name="s",
                                  num_cores=_NC, num_subcores=_NS)
    return pl.kernel(
        _sc_body,
        out_type=jax.ShapeDtypeStruct((_B,), jnp.float32),
        mesh=mesh,
        scratch_types=[
            pltpu.VMEM((_G, _CH, _SC), jnp.int32),
            pltpu.VMEM((_G, _CH, _SC), jnp.int32),
            pltpu.VMEM((2, 2, _SC, _D), jnp.bfloat16),
            pltpu.VMEM_SHARED((_V, _D), jnp.bfloat16),
            pltpu.VMEM((_G * _D,), jnp.float32),
            pltpu.VMEM((_G * _D,), jnp.float32),
            pltpu.VMEM((_RPW,), jnp.float32),
            pltpu.SemaphoreType.DMA,
            pltpu.SemaphoreType.DMA,
            pltpu.SemaphoreType.DMA,
            pltpu.SemaphoreType.DMA,
        ],
        compiler_params=pltpu.CompilerParams(use_tc_tiling_on_sc=False,
                                             needs_layout_passes=False),
        interpret=interpret,
    )


@jax.jit
def kernel(pred_ids, gt_ids, table):
    pred3 = pred_ids.astype(jnp.int32).reshape(_B, _CH, _SC)
    gt3 = gt_ids.astype(jnp.int32).reshape(_B, _CH, _SC)
    table_bf = table.astype(jnp.bfloat16)
    return _make_sc_kernel()(pred3, gt3, table_bf)


# speedup vs baseline: 1.3716x; 1.3716x over previous
import functools

import jax
import jax.numpy as jnp
from jax import lax
from jax.experimental import pallas as pl
from jax.experimental.pallas import tpu as pltpu
from jax.experimental.pallas import tpu_sc as plsc

_V = 10000
_D = 256
_B = 16384
_S = 200

_NC, _NS, _L = 2, 16, 16
_NW = _NC * _NS
_RPW = _B // _NW
_G = 16
_NG = _RPW // _G
_CH = 2
_SC = _S // _CH
_DV = _D // _L
_PK = _D // (2 * _L)
_TCH = 5


def _rsqrt_nr(x):
    xi = plsc.bitcast(x, jnp.int32)
    yi = jnp.int32(0x5F3759DF) - (xi >> 1)
    y = plsc.bitcast(yi, jnp.float32)
    for _ in range(3):
        y = y * (1.5 - 0.5 * x * y * y)
    return y


_SCHED = ((0, 0), (1, 0), (0, 1), (1, 1))


def _sc_body(pred_hbm, gt_hbm, table_hbm, out_hbm,
             idx_p, idx_g, bufs, tshared, sums_p, sums_g, rewards,
             sem0, sem1):
    sid = lax.axis_index("s")
    wid = sid * _NC + lax.axis_index("c")
    base = wid * _RPW
    zero = jnp.zeros((_L,), jnp.float32)
    rows16 = lax.iota(jnp.int32, _L) * _D
    sems = (sem0, sem1)
    idxs = (idx_p, idx_g)

    @pl.when(sid == 0)
    def _():
        pltpu.sync_copy(table_hbm, tshared)
    plsc.subcore_barrier()

    def chunk_copy(i, side, half, par):
        return pltpu.make_async_copy(
            tshared.at[idxs[side].at[i, half]], bufs.at[par], sems[par])

    def reduce_chunk(par, accs, side):
        def run(jj, accs_):
            f = list(accs_)
            j0 = jj * _TCH
            for k in range(_PK):
                b = bufs[par, j0, pl.ds(k * 2 * _L, 2 * _L)]
                for t in range(1, _TCH):
                    b = b + bufs[par, j0 + t, pl.ds(k * 2 * _L, 2 * _L)]
                lo, hi = plsc.unpack(b, format=plsc.PackFormat.INTERLEAVED)
                f[side * _DV + 2 * k] += lo
                f[side * _DV + 2 * k + 1] += hi
            return tuple(f)

        return lax.fori_loop(0, _SC // _TCH, run, accs)

    def group_body(g, carry):
        rbase = base + g * _G
        pltpu.sync_copy(pred_hbm.at[pl.ds(rbase, _G)], idx_p)
        pltpu.sync_copy(gt_hbm.at[pl.ds(rbase, _G)], idx_g)
        chunk_copy(0, _SCHED[0][0], _SCHED[0][1], 0).start()

        def row_body(i, c2):
            accs = (zero,) * (2 * _DV)
            for c, (side, half) in enumerate(_SCHED):
                par = c % 2
                if c + 1 < len(_SCHED):
                    ns, nh = _SCHED[c + 1]
                    chunk_copy(i, ns, nh, 1 - par).start()
                else:
                    @pl.when(i + 1 < _G)
                    def _():
                        chunk_copy(i + 1, _SCHED[0][0], _SCHED[0][1],
                                   1 - par).start()
                chunk_copy(i, side, half, par).wait()
                accs = reduce_chunk(par, accs, side)
            for k in range(_DV):
                sums_p[pl.ds(i * _D + k * _L, _L)] = accs[k]
                sums_g[pl.ds(i * _D + k * _L, _L)] = accs[_DV + k]
            return c2

        lax.fori_loop(0, _G, row_body, 0)

        def fin(d, carry3):
            dot, np_, ng_ = carry3
            idxv = rows16 + d
            p = plsc.load_gather(sums_p, [idxv])
            q = plsc.load_gather(sums_g, [idxv])
            return dot + p * q, np_ + p * p, ng_ + q * q

        dot, np_, ng_ = lax.fori_loop(0, _D, fin, (zero, zero, zero))
        inv2 = jnp.float32(1.0 / (_S * _S))
        np_m = jnp.maximum(np_ * inv2, 1e-16)
        ng_m = jnp.maximum(ng_ * inv2, 1e-16)
        sim = dot * inv2 * _rsqrt_nr(np_m * ng_m)
        rewards[pl.ds(g * _G, _G)] = 0.7 * jnp.maximum(sim, 0.0)
        return carry

    lax.fori_loop(0, _NG, group_body, 0)
    pltpu.sync_copy(rewards, out_hbm.at[pl.ds(base, _RPW)])


def _make_sc_kernel(interpret=False):
    mesh = plsc.VectorSubcoreMesh(core_axis_name="c", subcore_axis_name="s",
                                  num_cores=_NC, num_subcores=_NS)
    return pl.kernel(
        _sc_body,
        out_type=jax.ShapeDtypeStruct((_B,), jnp.float32),
        mesh=mesh,
        scratch_types=[
            pltpu.VMEM((_G, _CH, _SC), jnp.int32),
            pltpu.VMEM((_G, _CH, _SC), jnp.int32),
            pltpu.VMEM((2, _SC, _D), jnp.bfloat16),
            pltpu.VMEM_SHARED((_V, _D), jnp.bfloat16),
            pltpu.VMEM((_G * _D,), jnp.float32),
            pltpu.VMEM((_G * _D,), jnp.float32),
            pltpu.VMEM((_RPW,), jnp.float32),
            pltpu.SemaphoreType.DMA,
            pltpu.SemaphoreType.DMA,
        ],
        compiler_params=pltpu.CompilerParams(use_tc_tiling_on_sc=False,
                                             needs_layout_passes=False),
        interpret=interpret,
    )


@jax.jit
def kernel(pred_ids, gt_ids, table):
    pred3 = pred_ids.astype(jnp.int32).reshape(_B, _CH, _SC)
    gt3 = gt_ids.astype(jnp.int32).reshape(_B, _CH, _SC)
    table_bf = table.astype(jnp.bfloat16)
    return _make_sc_kernel()(pred3, gt3, table_bf)


# speedup vs baseline: 1.3878x; 1.0118x over previous
import functools

import jax
import jax.numpy as jnp
from jax import lax
from jax.experimental import pallas as pl
from jax.experimental.pallas import tpu as pltpu
from jax.experimental.pallas import tpu_sc as plsc

_V = 10000
_D = 256
_B = 16384
_S = 200

_NC, _NS, _L = 2, 16, 16
_NW = _NC * _NS
_RPW = _B // _NW
_G = 16
_NG = _RPW // _G
_CH = 2
_SC = _S // _CH
_DV = _D // _L
_PK = _D // (2 * _L)
_TCH = 5


def _rsqrt_nr(x):
    xi = plsc.bitcast(x, jnp.int32)
    yi = jnp.int32(0x5F3759DF) - (xi >> 1)
    y = plsc.bitcast(yi, jnp.float32)
    for _ in range(3):
        y = y * (1.5 - 0.5 * x * y * y)
    return y


_SCHED = ((0, 0), (1, 0), (0, 1), (1, 1))


def _sc_body(pred_hbm, gt_hbm, table_hbm, out_hbm,
             idx_p, idx_g, bufs, tshared, sums_p, sums_g, rewards,
             sem0, sem1):
    sid = lax.axis_index("s")
    wid = sid * _NC + lax.axis_index("c")
    base = wid * _RPW
    zero = jnp.zeros((_L,), jnp.float32)
    rows16 = lax.iota(jnp.int32, _L) * _D
    sems = (sem0, sem1)
    idxs = (idx_p, idx_g)

    @pl.when(sid == 0)
    def _():
        pltpu.sync_copy(table_hbm, tshared)
    plsc.subcore_barrier()

    def chunk_copy(i, side, half, par):
        return pltpu.make_async_copy(
            tshared.at[idxs[side].at[i, half]], bufs.at[par], sems[par])

    def reduce_chunk(par, accs, side):
        def run(jj, accs_):
            f = list(accs_)
            j0 = jj * _TCH
            for k in range(_PK):
                b = bufs[par, j0, pl.ds(k * 2 * _L, 2 * _L)]
                for t in range(1, _TCH):
                    b = b + bufs[par, j0 + t, pl.ds(k * 2 * _L, 2 * _L)]
                lo, hi = plsc.unpack(b, format=plsc.PackFormat.INTERLEAVED)
                f[side * _DV + 2 * k] += lo
                f[side * _DV + 2 * k + 1] += hi
            return tuple(f)

        return plsc.parallel_loop(0, _SC // _TCH, 1, unroll=2,
                                  carry=accs)(run)

    def stage_group(g):
        rbase = base + g * _G
        pltpu.sync_copy(pred_hbm.at[pl.ds(rbase, _G)], idx_p)
        pltpu.sync_copy(gt_hbm.at[pl.ds(rbase, _G)], idx_g)
        chunk_copy(0, _SCHED[0][0], _SCHED[0][1], 0).start()

    stage_group(0)

    def group_body(g, carry):
        def row_body(i, c2):
            accs = (zero,) * (2 * _DV)
            for c, (side, half) in enumerate(_SCHED):
                par = c % 2
                if c + 1 < len(_SCHED):
                    ns, nh = _SCHED[c + 1]
                    chunk_copy(i, ns, nh, 1 - par).start()
                else:
                    @pl.when(i + 1 < _G)
                    def _():
                        chunk_copy(i + 1, _SCHED[0][0], _SCHED[0][1],
                                   1 - par).start()
                chunk_copy(i, side, half, par).wait()
                accs = reduce_chunk(par, accs, side)
            for k in range(_DV):
                sums_p[pl.ds(i * _D + k * _L, _L)] = accs[k]
                sums_g[pl.ds(i * _D + k * _L, _L)] = accs[_DV + k]
            return c2

        lax.fori_loop(0, _G, row_body, 0)

        @pl.when(g + 1 < _NG)
        def _():
            stage_group(g + 1)

        def fin(d, carry3):
            dot, np_, ng_ = carry3
            idxv = rows16 + d
            p = plsc.load_gather(sums_p, [idxv])
            q = plsc.load_gather(sums_g, [idxv])
            return dot + p * q, np_ + p * p, ng_ + q * q

        dot, np_, ng_ = lax.fori_loop(0, _D, fin, (zero, zero, zero))
        inv2 = jnp.float32(1.0 / (_S * _S))
        np_m = jnp.maximum(np_ * inv2, 1e-16)
        ng_m = jnp.maximum(ng_ * inv2, 1e-16)
        sim = dot * inv2 * _rsqrt_nr(np_m * ng_m)
        rewards[pl.ds(g * _G, _G)] = 0.7 * jnp.maximum(sim, 0.0)
        return carry

    lax.fori_loop(0, _NG, group_body, 0)
    pltpu.sync_copy(rewards, out_hbm.at[pl.ds(base, _RPW)])


def _make_sc_kernel(interpret=False):
    mesh = plsc.VectorSubcoreMesh(core_axis_name="c", subcore_axis_name="s",
                                  num_cores=_NC, num_subcores=_NS)
    return pl.kernel(
        _sc_body,
        out_type=jax.ShapeDtypeStruct((_B,), jnp.float32),
        mesh=mesh,
        scratch_types=[
            pltpu.VMEM((_G, _CH, _SC), jnp.int32),
            pltpu.VMEM((_G, _CH, _SC), jnp.int32),
            pltpu.VMEM((2, _SC, _D), jnp.bfloat16),
            pltpu.VMEM_SHARED((_V, _D), jnp.bfloat16),
            pltpu.VMEM((_G * _D,), jnp.float32),
            pltpu.VMEM((_G * _D,), jnp.float32),
            pltpu.VMEM((_RPW,), jnp.float32),
            pltpu.SemaphoreType.DMA,
            pltpu.SemaphoreType.DMA,
        ],
        compiler_params=pltpu.CompilerParams(use_tc_tiling_on_sc=False,
                                             needs_layout_passes=False),
        interpret=interpret,
    )


@jax.jit
def kernel(pred_ids, gt_ids, table):
    pred3 = pred_ids.astype(jnp.int32).reshape(_B, _CH, _SC)
    gt3 = gt_ids.astype(jnp.int32).reshape(_B, _CH, _SC)
    table_bf = table.astype(jnp.bfloat16)
    return _make_sc_kernel()(pred3, gt3, table_bf)


# speedup vs baseline: 1.4788x; 1.0656x over previous
import functools

import jax
import jax.numpy as jnp
from jax import lax
from jax.experimental import pallas as pl
from jax.experimental.pallas import tpu as pltpu
from jax.experimental.pallas import tpu_sc as plsc

_V = 10000
_D = 256
_B = 16384
_S = 200

_NC, _NS, _L = 2, 16, 16
_NW = _NC * _NS
_RPW = _B // _NW
_G = 16
_NG = _RPW // _G
_SCA, _SCB = 120, 80
_HALVES = ((0, _SCA), (_SCA, _SCB))
_DV = _D // _L
_PK = _D // (2 * _L)
_TCH = 5


def _rsqrt_nr(x):
    xi = plsc.bitcast(x, jnp.int32)
    yi = jnp.int32(0x5F3759DF) - (xi >> 1)
    y = plsc.bitcast(yi, jnp.float32)
    for _ in range(3):
        y = y * (1.5 - 0.5 * x * y * y)
    return y


_SCHED = ((0, 0), (1, 0), (0, 1), (1, 1))


def _sc_body(pred_hbm, gt_hbm, table_hbm, out_hbm,
             idx_p, idx_g, bufs, tshared, sums_p, sums_g, rewards,
             sem0, sem1):
    sid = lax.axis_index("s")
    wid = sid * _NC + lax.axis_index("c")
    base = wid * _RPW
    zero = jnp.zeros((_L,), jnp.float32)
    rows16 = lax.iota(jnp.int32, _L) * _D
    sems = (sem0, sem1)
    idxs = (idx_p, idx_g)

    @pl.when(sid == 0)
    def _():
        pltpu.sync_copy(table_hbm, tshared)
    plsc.subcore_barrier()

    def chunk_copy(i, side, half, par):
        off, n = _HALVES[half]
        dst = bufs.at[par] if n == _SCA else bufs.at[par, pl.ds(0, n)]
        return pltpu.make_async_copy(
            tshared.at[idxs[side].at[i, pl.ds(off, n)]], dst, sems[par])

    def reduce_chunk(par, accs, side, ntok):
        def run(jj, accs_):
            f = list(accs_)
            j0 = jj * _TCH
            for k in range(_PK):
                b = bufs[par, j0, pl.ds(k * 2 * _L, 2 * _L)]
                for t in range(1, _TCH):
                    b = b + bufs[par, j0 + t, pl.ds(k * 2 * _L, 2 * _L)]
                lo, hi = plsc.unpack(b, format=plsc.PackFormat.INTERLEAVED)
                f[side * _DV + 2 * k] += lo
                f[side * _DV + 2 * k + 1] += hi
            return tuple(f)

        return plsc.parallel_loop(0, ntok // _TCH, 1, unroll=2,
                                  carry=accs)(run)

    def stage_group(g):
        rbase = base + g * _G
        pltpu.sync_copy(pred_hbm.at[pl.ds(rbase, _G)], idx_p)
        pltpu.sync_copy(gt_hbm.at[pl.ds(rbase, _G)], idx_g)
        chunk_copy(0, _SCHED[0][0], _SCHED[0][1], 0).start()

    stage_group(0)

    def group_body(g, carry):
        def row_body(i, c2):
            accs = (zero,) * (2 * _DV)
            for c, (side, half) in enumerate(_SCHED):
                par = c % 2
                if c + 1 < len(_SCHED):
                    ns, nh = _SCHED[c + 1]
                    chunk_copy(i, ns, nh, 1 - par).start()
                else:
                    @pl.when(i + 1 < _G)
                    def _():
                        chunk_copy(i + 1, _SCHED[0][0], _SCHED[0][1],
                                   1 - par).start()
                chunk_copy(i, side, half, par).wait()
                accs = reduce_chunk(par, accs, side, _HALVES[half][1])
            for k in range(_DV):
                sums_p[pl.ds(i * _D + k * _L, _L)] = accs[k]
                sums_g[pl.ds(i * _D + k * _L, _L)] = accs[_DV + k]
            return c2

        lax.fori_loop(0, _G, row_body, 0)

        @pl.when(g + 1 < _NG)
        def _():
            stage_group(g + 1)

        def fin(d, carry3):
            dot, np_, ng_ = carry3
            idxv = rows16 + d
            p = plsc.load_gather(sums_p, [idxv])
            q = plsc.load_gather(sums_g, [idxv])
            return dot + p * q, np_ + p * p, ng_ + q * q

        dot, np_, ng_ = lax.fori_loop(0, _D, fin, (zero, zero, zero))
        inv2 = jnp.float32(1.0 / (_S * _S))
        np_m = jnp.maximum(np_ * inv2, 1e-16)
        ng_m = jnp.maximum(ng_ * inv2, 1e-16)
        sim = dot * inv2 * _rsqrt_nr(np_m * ng_m)
        rewards[pl.ds(g * _G, _G)] = 0.7 * jnp.maximum(sim, 0.0)
        return carry

    lax.fori_loop(0, _NG, group_body, 0)
    pltpu.sync_copy(rewards, out_hbm.at[pl.ds(base, _RPW)])


def _make_sc_kernel(interpret=False):
    mesh = plsc.VectorSubcoreMesh(core_axis_name="c", subcore_axis_name="s",
                                  num_cores=_NC, num_subcores=_NS)
    return pl.kernel(
        _sc_body,
        out_type=jax.ShapeDtypeStruct((_B,), jnp.float32),
        mesh=mesh,
        scratch_types=[
            pltpu.VMEM((_G, _S), jnp.int32),
            pltpu.VMEM((_G, _S), jnp.int32),
            pltpu.VMEM((2, _SCA, _D), jnp.bfloat16),
            pltpu.VMEM_SHARED((_V, _D), jnp.bfloat16),
            pltpu.VMEM((_G * _D,), jnp.float32),
            pltpu.VMEM((_G * _D,), jnp.float32),
            pltpu.VMEM((_RPW,), jnp.float32),
            pltpu.SemaphoreType.DMA,
            pltpu.SemaphoreType.DMA,
        ],
        compiler_params=pltpu.CompilerParams(use_tc_tiling_on_sc=False,
                                             needs_layout_passes=False),
        interpret=interpret,
    )


@jax.jit
def kernel(pred_ids, gt_ids, table):
    table_bf = table.astype(jnp.bfloat16)
    return _make_sc_kernel()(pred_ids, gt_ids, table_bf)


# speedup vs baseline: 1.5568x; 1.0528x over previous
import functools

import jax
import jax.numpy as jnp
from jax import lax
from jax.experimental import pallas as pl
from jax.experimental.pallas import tpu as pltpu
from jax.experimental.pallas import tpu_sc as plsc

_V = 10000
_D = 256
_B = 16384
_S = 200

_NC, _NS, _L = 2, 16, 16
_NW = _NC * _NS
_RPW = _B // _NW
_G = 16
_NG = _RPW // _G
_PIECES = ((0, 72), (72, 64), (136, 64))
_SCA = 72
_DV = _D // _L
_PK = _D // (2 * _L)
_TCH = 4


def _rsqrt_nr(x):
    xi = plsc.bitcast(x, jnp.int32)
    yi = jnp.int32(0x5F3759DF) - (xi >> 1)
    y = plsc.bitcast(yi, jnp.float32)
    for _ in range(3):
        y = y * (1.5 - 0.5 * x * y * y)
    return y


_SCHED = ((0, 0), (1, 0), (0, 1), (1, 1), (0, 2), (1, 2))


def _sc_body(pred_hbm, gt_hbm, table_hbm, out_hbm,
             idx_p, idx_g, bufs, tshared, sums_p, sums_g, rewards,
             sem0, sem1, sem2):
    sid = lax.axis_index("s")
    wid = sid * _NC + lax.axis_index("c")
    base = wid * _RPW
    zero = jnp.zeros((_L,), jnp.float32)
    rows16 = lax.iota(jnp.int32, _L) * _D
    sems = (sem0, sem1, sem2)
    idxs = (idx_p, idx_g)

    @pl.when(sid == 0)
    def _():
        pltpu.sync_copy(table_hbm, tshared)
    plsc.subcore_barrier()

    def chunk_copy(i, c):
        side, piece = _SCHED[c % 6]
        slot = c % 3
        off, n = _PIECES[piece]
        dst = bufs.at[slot] if n == _SCA else bufs.at[slot, pl.ds(0, n)]
        return pltpu.make_async_copy(
            tshared.at[idxs[side].at[i, pl.ds(off, n)]], dst, sems[slot])

    def reduce_chunk(par, accs, side, ntok):
        def run(jj, accs_):
            f = list(accs_)
            j0 = jj * _TCH
            for k in range(_PK):
                b = bufs[par, j0, pl.ds(k * 2 * _L, 2 * _L)]
                for t in range(1, _TCH):
                    b = b + bufs[par, j0 + t, pl.ds(k * 2 * _L, 2 * _L)]
                lo, hi = plsc.unpack(b, format=plsc.PackFormat.INTERLEAVED)
                f[side * _DV + 2 * k] += lo
                f[side * _DV + 2 * k + 1] += hi
            return tuple(f)

        return plsc.parallel_loop(0, ntok // _TCH, 1, unroll=2,
                                  carry=accs)(run)

    def stage_group(g):
        rbase = base + g * _G
        pltpu.sync_copy(pred_hbm.at[pl.ds(rbase, _G)], idx_p)
        pltpu.sync_copy(gt_hbm.at[pl.ds(rbase, _G)], idx_g)
        chunk_copy(0, 0).start()
        chunk_copy(0, 1).start()

    stage_group(0)

    def group_body(g, carry):
        def row_body(i, c2):
            accs = (zero,) * (2 * _DV)
            for c in range(6):
                if c + 2 < 6:
                    chunk_copy(i, c + 2).start()
                else:
                    @pl.when(i + 1 < _G)
                    def _():
                        chunk_copy(i + 1, c + 2 - 6).start()
                chunk_copy(i, c).wait()
                side, piece = _SCHED[c]
                accs = reduce_chunk(c % 3, accs, side, _PIECES[piece][1])
            for k in range(_DV):
                sums_p[pl.ds(i * _D + k * _L, _L)] = accs[k]
                sums_g[pl.ds(i * _D + k * _L, _L)] = accs[_DV + k]
            return c2

        lax.fori_loop(0, _G, row_body, 0)

        @pl.when(g + 1 < _NG)
        def _():
            stage_group(g + 1)

        def fin(d, carry3):
            dot, np_, ng_ = carry3
            idxv = rows16 + d
            p = plsc.load_gather(sums_p, [idxv])
            q = plsc.load_gather(sums_g, [idxv])
            return dot + p * q, np_ + p * p, ng_ + q * q

        dot, np_, ng_ = lax.fori_loop(0, _D, fin, (zero, zero, zero))
        inv2 = jnp.float32(1.0 / (_S * _S))
        np_m = jnp.maximum(np_ * inv2, 1e-16)
        ng_m = jnp.maximum(ng_ * inv2, 1e-16)
        sim = dot * inv2 * _rsqrt_nr(np_m * ng_m)
        rewards[pl.ds(g * _G, _G)] = 0.7 * jnp.maximum(sim, 0.0)
        return carry

    lax.fori_loop(0, _NG, group_body, 0)
    pltpu.sync_copy(rewards, out_hbm.at[pl.ds(base, _RPW)])


def _make_sc_kernel(interpret=False):
    mesh = plsc.VectorSubcoreMesh(core_axis_name="c", subcore_axis_name="s",
                                  num_cores=_NC, num_subcores=_NS)
    return pl.kernel(
        _sc_body,
        out_type=jax.ShapeDtypeStruct((_B,), jnp.float32),
        mesh=mesh,
        scratch_types=[
            pltpu.VMEM((_G, _S), jnp.int32),
            pltpu.VMEM((_G, _S), jnp.int32),
            pltpu.VMEM((3, _SCA, _D), jnp.bfloat16),
            pltpu.VMEM_SHARED((_V, _D), jnp.bfloat16),
            pltpu.VMEM((_G * _D,), jnp.float32),
            pltpu.VMEM((_G * _D,), jnp.float32),
            pltpu.VMEM((_RPW,), jnp.float32),
            pltpu.SemaphoreType.DMA,
            pltpu.SemaphoreType.DMA,
            pltpu.SemaphoreType.DMA,
        ],
        compiler_params=pltpu.CompilerParams(use_tc_tiling_on_sc=False,
                                             needs_layout_passes=False),
        interpret=interpret,
    )


@jax.jit
def kernel(pred_ids, gt_ids, table):
    table_bf = table.astype(jnp.bfloat16)
    return _make_sc_kernel()(pred_ids, gt_ids, table_bf)


# speedup vs baseline: 1.5581x; 1.0008x over previous
import functools

import jax
import jax.numpy as jnp
from jax import lax
from jax.experimental import pallas as pl
from jax.experimental.pallas import tpu as pltpu
from jax.experimental.pallas import tpu_sc as plsc

_V = 10000
_D = 256
_B = 16384
_S = 200

_NC, _NS, _L = 2, 16, 16
_NW = _NC * _NS
_RPW = _B // _NW
_G = 16
_NG = _RPW // _G
_PIECES = ((0, 56), (56, 48), (104, 48), (152, 48))
_SCA = 56
_DV = _D // _L
_PK = _D // (2 * _L)
_TCH = 4


def _rsqrt_nr(x):
    xi = plsc.bitcast(x, jnp.int32)
    yi = jnp.int32(0x5F3759DF) - (xi >> 1)
    y = plsc.bitcast(yi, jnp.float32)
    for _ in range(3):
        y = y * (1.5 - 0.5 * x * y * y)
    return y


_SCHED = ((0, 0), (1, 0), (0, 1), (1, 1), (0, 2), (1, 2), (0, 3), (1, 3))


def _sc_body(pred_hbm, gt_hbm, table_hbm, out_hbm,
             idx_p, idx_g, bufs, tshared, sums_p, sums_g, rewards,
             sem0, sem1, sem2, sem3):
    sid = lax.axis_index("s")
    wid = sid * _NC + lax.axis_index("c")
    base = wid * _RPW
    zero = jnp.zeros((_L,), jnp.float32)
    rows16 = lax.iota(jnp.int32, _L) * _D
    sems = (sem0, sem1, sem2, sem3)
    idxs = (idx_p, idx_g)

    @pl.when(sid == 0)
    def _():
        pltpu.sync_copy(table_hbm, tshared)
    plsc.subcore_barrier()

    def chunk_copy(i, c):
        side, piece = _SCHED[c % 8]
        slot = c % 4
        off, n = _PIECES[piece]
        dst = bufs.at[slot] if n == _SCA else bufs.at[slot, pl.ds(0, n)]
        return pltpu.make_async_copy(
            tshared.at[idxs[side].at[i, pl.ds(off, n)]], dst, sems[slot])

    def reduce_chunk(par, accs, side, ntok):
        def run(jj, accs_):
            f = list(accs_)
            j0 = jj * _TCH
            for k in range(_PK):
                b = bufs[par, j0, pl.ds(k * 2 * _L, 2 * _L)]
                for t in range(1, _TCH):
                    b = b + bufs[par, j0 + t, pl.ds(k * 2 * _L, 2 * _L)]
                lo, hi = plsc.unpack(b, format=plsc.PackFormat.INTERLEAVED)
                f[side * _DV + 2 * k] += lo
                f[side * _DV + 2 * k + 1] += hi
            return tuple(f)

        return plsc.parallel_loop(0, ntok // _TCH, 1, unroll=2,
                                  carry=accs)(run)

    def stage_group(g):
        rbase = base + g * _G
        pltpu.sync_copy(pred_hbm.at[pl.ds(rbase, _G)], idx_p)
        pltpu.sync_copy(gt_hbm.at[pl.ds(rbase, _G)], idx_g)
        chunk_copy(0, 0).start()
        chunk_copy(0, 1).start()
        chunk_copy(0, 2).start()

    stage_group(0)

    def group_body(g, carry):
        def row_body(i, c2):
            accs = (zero,) * (2 * _DV)
            for c in range(8):
                if c + 3 < 8:
                    chunk_copy(i, c + 3).start()
                else:
                    @pl.when(i + 1 < _G)
                    def _():
                        chunk_copy(i + 1, c + 3 - 8).start()
                chunk_copy(i, c).wait()
                side, piece = _SCHED[c]
                accs = reduce_chunk(c % 4, accs, side, _PIECES[piece][1])
            for k in range(_DV):
                sums_p[pl.ds(i * _D + k * _L, _L)] = accs[k]
                sums_g[pl.ds(i * _D + k * _L, _L)] = accs[_DV + k]
            return c2

        lax.fori_loop(0, _G, row_body, 0)

        @pl.when(g + 1 < _NG)
        def _():
            stage_group(g + 1)

        def fin(d, carry3):
            dot, np_, ng_ = carry3
            idxv = rows16 + d
            p = plsc.load_gather(sums_p, [idxv])
            q = plsc.load_gather(sums_g, [idxv])
            return dot + p * q, np_ + p * p, ng_ + q * q

        dot, np_, ng_ = lax.fori_loop(0, _D, fin, (zero, zero, zero))
        inv2 = jnp.float32(1.0 / (_S * _S))
        np_m = jnp.maximum(np_ * inv2, 1e-16)
        ng_m = jnp.maximum(ng_ * inv2, 1e-16)
        sim = dot * inv2 * _rsqrt_nr(np_m * ng_m)
        rewards[pl.ds(g * _G, _G)] = 0.7 * jnp.maximum(sim, 0.0)
        return carry

    lax.fori_loop(0, _NG, group_body, 0)
    pltpu.sync_copy(rewards, out_hbm.at[pl.ds(base, _RPW)])


def _make_sc_kernel(interpret=False):
    mesh = plsc.VectorSubcoreMesh(core_axis_name="c", subcore_axis_name="s",
                                  num_cores=_NC, num_subcores=_NS)
    return pl.kernel(
        _sc_body,
        out_type=jax.ShapeDtypeStruct((_B,), jnp.float32),
        mesh=mesh,
        scratch_types=[
            pltpu.VMEM((_G, _S), jnp.int32),
            pltpu.VMEM((_G, _S), jnp.int32),
            pltpu.VMEM((4, _SCA, _D), jnp.bfloat16),
            pltpu.VMEM_SHARED((_V, _D), jnp.bfloat16),
            pltpu.VMEM((_G * _D,), jnp.float32),
            pltpu.VMEM((_G * _D,), jnp.float32),
            pltpu.VMEM((_RPW,), jnp.float32),
            pltpu.SemaphoreType.DMA,
            pltpu.SemaphoreType.DMA,
            pltpu.SemaphoreType.DMA,
            pltpu.SemaphoreType.DMA,
        ],
        compiler_params=pltpu.CompilerParams(use_tc_tiling_on_sc=False,
                                             needs_layout_passes=False),
        interpret=interpret,
    )


@jax.jit
def kernel(pred_ids, gt_ids, table):
    table_bf = table.astype(jnp.bfloat16)
    return _make_sc_kernel()(pred_ids, gt_ids, table_bf)
